# baseline (speedup 1.0000x reference)
import functools

import jax
import jax.numpy as jnp
from jax import lax
from jax._src import core as _jax_core
from jax._src.pallas import core as _pallas_core
from jax.experimental import pallas as pl
from jax.experimental.pallas import tpu as pltpu
from jax.experimental.pallas import tpu_sc as plsc

VOCAB = 1000000
EMBED = 64
BATCH = 16384

_CHUNK = 128


def _to_default_space(x):
  return _pallas_core.with_memory_space_constraint_p.bind(
      x, memory_space=_jax_core.MemorySpace.Device)


def _build_kernel(name):
  info = plsc.get_sparse_core_info()
  nc, ns = info.num_cores, info.num_subcores
  nw = nc * ns
  b_per_w = BATCH // nw
  n_chunks = b_per_w // _CHUNK

  mesh = plsc.VectorSubcoreMesh(core_axis_name="c", subcore_axis_name="s")

  @functools.partial(
      pl.kernel,
      mesh=mesh,
      name=name,
      compiler_params=pltpu.CompilerParams(use_tc_tiling_on_sc=False),
      out_type=pltpu.HBM((BATCH, EMBED), jnp.float32),
      scratch_types=[
          pltpu.VMEM((n_chunks, _CHUNK), jnp.int32),
          pltpu.VMEM((b_per_w, EMBED), jnp.float32),
          pltpu.SemaphoreType.DMA,
      ],
  )
  def lookup(idx_hbm, table_hbm, out, idx_v, rows_v, sem):
    wid = lax.axis_index("s") * nc + lax.axis_index("c")
    base = wid * b_per_w

    pltpu.sync_copy(idx_hbm.at[pl.ds(wid * n_chunks, n_chunks)], idx_v)

    copies = []
    for j in range(n_chunks):
      copies.append(pltpu.async_copy(
          table_hbm.at[idx_v.at[j]],
          rows_v.at[pl.ds(j * _CHUNK, _CHUNK)], sem))
    for c in copies:
      c.wait()

    pltpu.sync_copy(rows_v, out.at[pl.ds(base, b_per_w)])

  return lookup


_lookup_word = _build_kernel("word_lookup")
_lookup_ctx = _build_kernel("ctx_lookup")


@jax.jit
def kernel(center, context, word_embeds, context_embeds):
  c2 = center.astype(jnp.int32).reshape(-1, _CHUNK)
  x2 = context.astype(jnp.int32).reshape(-1, _CHUNK)
  out_c = _lookup_word(c2, word_embeds)
  out_x = _lookup_ctx(x2, context_embeds)
  return _to_default_space(out_c), _to_default_space(out_x)


# speedup vs baseline: 1.2243x; 1.2243x over previous
import functools

import jax
import jax.numpy as jnp
from jax import lax
from jax._src import core as _jax_core
from jax._src.pallas import core as _pallas_core
from jax.experimental import pallas as pl
from jax.experimental.pallas import tpu as pltpu
from jax.experimental.pallas import tpu_sc as plsc

VOCAB = 1000000
EMBED = 64
BATCH = 16384

_PAIRW = 2 * EMBED
_CHUNK = 128
_HALF = 256


def _to_default_space(x):
  return _pallas_core.with_memory_space_constraint_p.bind(
      x, memory_space=_jax_core.MemorySpace.Device)


def _sc_info():
  info = plsc.get_sparse_core_info()
  return info.num_cores, info.num_subcores


def _build_pair_gather():
  nc, ns = _sc_info()
  nw = nc * ns
  b_per_w = BATCH // nw
  n_chunks = b_per_w // _CHUNK

  mesh = plsc.VectorSubcoreMesh(core_axis_name="c", subcore_axis_name="s")

  @functools.partial(
      pl.kernel,
      mesh=mesh,
      name="word_pair_gather",
      compiler_params=pltpu.CompilerParams(needs_layout_passes=False),
      out_type=pltpu.HBM((BATCH, EMBED), jnp.float32),
      scratch_types=[
          pltpu.VMEM((b_per_w,), jnp.int32),
          pltpu.VMEM((n_chunks, _CHUNK), jnp.int32),
          pltpu.VMEM((2, _CHUNK, _PAIRW), jnp.float32),
          pltpu.VMEM((2, _CHUNK, EMBED), jnp.float32),
          pltpu.SemaphoreType.DMA,
      ],
  )
  def pair_gather(idx_hbm, pairs_hbm, out, idx_v, pid_v, stg_v, rows_v, sem):
    wid = lax.axis_index("s") * nc + lax.axis_index("c")
    base = wid * b_per_w
    lanes = lax.iota(jnp.int32, 16)

    pltpu.sync_copy(idx_hbm.at[pl.ds(base, b_per_w)], idx_v)

    def pid_body(g, _):
      iv = idx_v[pl.ds(g * 16, 16)]
      pid_v[g // 8, pl.ds((g % 8) * 16, 16)] = lax.shift_right_logical(iv, 1)
      return 0

    lax.fori_loop(0, b_per_w // 16, pid_body, 0)

    def fire(j, b):
      pltpu.async_copy(pairs_hbm.at[pid_v.at[j]], stg_v.at[b], sem)

    def drain(b):
      pltpu.make_async_copy(pairs_hbm.at[pid_v.at[0]], stg_v.at[b],
                            sem).wait()

    def extract(j, b):
      bsel = jnp.full((16,), b, jnp.int32)

      def group_body(g, _):
        kvec = g * 16 + lanes
        sub = lax.bitwise_and(idx_v[pl.ds(j * _CHUNK + g * 16, 16)],
                              jnp.int32(1))
        colbase = sub * EMBED

        def col_body(col, _):
          cv = jnp.full((16,), col, jnp.int32)
          vals = plsc.load_gather(stg_v, [bsel, kvec, colbase + cv])
          plsc.store_scatter(rows_v, [bsel, kvec, cv], vals)
          return 0

        lax.fori_loop(0, EMBED, col_body, 0)
        return 0

      lax.fori_loop(0, _CHUNK // 16, group_body, 0)

    for j in range(n_chunks):
      b = j % 2
      if j == 0:
        fire(0, 0)
      drain(b)
      if j + 1 < n_chunks:
        fire(j + 1, 1 - b)
      extract(j, b)
      pltpu.sync_copy(rows_v.at[b],
                      out.at[pl.ds(base + j * _CHUNK, _CHUNK)])

  return pair_gather


def _build_row_gather():
  nc, ns = _sc_info()
  nw = nc * ns
  b_per_w = BATCH // nw
  n_halves = b_per_w // _HALF

  mesh = plsc.VectorSubcoreMesh(core_axis_name="c", subcore_axis_name="s")

  @functools.partial(
      pl.kernel,
      mesh=mesh,
      name="ctx_row_gather",
      out_type=pltpu.HBM((BATCH, EMBED), jnp.float32),
      scratch_types=[
          pltpu.VMEM((b_per_w,), jnp.int32),
          pltpu.VMEM((_HALF, EMBED), jnp.float32),
          pltpu.SemaphoreType.DMA,
      ],
  )
  def row_gather(idx_hbm, table_hbm, out, idx_v, rows_v, sem):
    wid = lax.axis_index("s") * nc + lax.axis_index("c")
    base = wid * b_per_w

    pltpu.sync_copy(idx_hbm.at[pl.ds(base, b_per_w)], idx_v)

    for half in range(n_halves):
      def group_body(g, _):
        iv = idx_v[pl.ds(half * _HALF + g * 16, 16)]
        for lane in range(16):
          pltpu.async_copy(table_hbm.at[pl.ds(iv[lane], 1)],
                           rows_v.at[pl.ds(g * 16 + lane, 1)], sem)
        return 0

      lax.fori_loop(0, _HALF // 16, group_body, 0)

      pltpu.make_async_copy(table_hbm.at[pl.ds(0, _HALF)], rows_v,
                            sem).wait()

      pltpu.sync_copy(rows_v, out.at[pl.ds(base + half * _HALF, _HALF)])

  return row_gather


_pair_gather = _build_pair_gather()
_row_gather = _build_row_gather()


@jax.jit
def kernel(center, context, word_embeds, context_embeds):
  c32 = center.astype(jnp.int32)
  x32 = context.astype(jnp.int32)
  out_x = _row_gather(x32, context_embeds)
  word_pairs = word_embeds.reshape(VOCAB // 2, _PAIRW)
  out_c = _pair_gather(c32, word_pairs)
  return _to_default_space(out_c), _to_default_space(out_x)


# speedup vs baseline: 1.5769x; 1.2880x over previous
import functools

import jax
import jax.numpy as jnp
from jax import lax
from jax._src import core as _jax_core
from jax._src.pallas import core as _pallas_core
from jax.experimental import pallas as pl
from jax.experimental.pallas import tpu as pltpu
from jax.experimental.pallas import tpu_sc as plsc

VOCAB = 1000000
EMBED = 64
BATCH = 16384

_HALF = 256


def _to_default_space(x):
  return _pallas_core.with_memory_space_constraint_p.bind(
      x, memory_space=_jax_core.MemorySpace.Device)


def _build_kernel():
  info = plsc.get_sparse_core_info()
  nc, ns = info.num_cores, info.num_subcores
  nw = nc * ns
  b_per_w = BATCH // nw
  n_halves = b_per_w // _HALF

  mesh = plsc.VectorSubcoreMesh(core_axis_name="c", subcore_axis_name="s")

  @functools.partial(
      pl.kernel,
      mesh=mesh,
      out_type=(
          pltpu.HBM((BATCH, EMBED), jnp.float32),
          pltpu.HBM((BATCH, EMBED), jnp.float32),
      ),
      scratch_types=[
          pltpu.VMEM((b_per_w,), jnp.int32),
          pltpu.VMEM((b_per_w,), jnp.int32),
          pltpu.VMEM((_HALF, EMBED), jnp.float32),
          pltpu.VMEM((_HALF, EMBED), jnp.float32),
          pltpu.SemaphoreType.DMA,
          pltpu.SemaphoreType.DMA,
      ],
  )
  def lookup(center_hbm, context_hbm, word_hbm, ctx_hbm,
             out_c, out_x, cidx_v, xidx_v, crows_v, xrows_v, sem_c, sem_x):
    wid = lax.axis_index("s") * nc + lax.axis_index("c")
    base = wid * b_per_w

    pltpu.sync_copy(center_hbm.at[pl.ds(base, b_per_w)], cidx_v)
    pltpu.sync_copy(context_hbm.at[pl.ds(base, b_per_w)], xidx_v)

    for half in range(n_halves):
      def group_body(g, _):
        cv = cidx_v[pl.ds(half * _HALF + g * 16, 16)]
        xv = xidx_v[pl.ds(half * _HALF + g * 16, 16)]
        for lane in range(16):
          pltpu.async_copy(word_hbm.at[pl.ds(cv[lane], 1)],
                           crows_v.at[pl.ds(g * 16 + lane, 1)], sem_c)
          pltpu.async_copy(ctx_hbm.at[pl.ds(xv[lane], 1)],
                           xrows_v.at[pl.ds(g * 16 + lane, 1)], sem_x)
        return 0

      lax.fori_loop(0, _HALF // 16, group_body, 0)

      pltpu.make_async_copy(word_hbm.at[pl.ds(0, _HALF)], crows_v,
                            sem_c).wait()
      pltpu.make_async_copy(ctx_hbm.at[pl.ds(0, _HALF)], xrows_v,
                            sem_x).wait()

      pltpu.sync_copy(crows_v, out_c.at[pl.ds(base + half * _HALF, _HALF)])
      pltpu.sync_copy(xrows_v, out_x.at[pl.ds(base + half * _HALF, _HALF)])

  return lookup


_lookup = _build_kernel()


@jax.jit
def kernel(center, context, word_embeds, context_embeds):
  out_c, out_x = _lookup(center.astype(jnp.int32), context.astype(jnp.int32),
                         word_embeds, context_embeds)
  return _to_default_space(out_c), _to_default_space(out_x)
